# baseline (speedup 1.0000x reference)
import functools

import jax
import jax.numpy as jnp
from jax import lax
from jax.experimental import pallas as pl
from jax.experimental.pallas import tpu as pltpu
from jax.experimental.pallas import tpu_sc as plsc

N_IN = 100000
N_OUT = 100000
KV = 27
S = 60000
CIN = 32
COUT = 64

E = KV * S
CH = 1024
SUB = CH // 128
N_CHUNKS = (E + CH - 1) // CH
E_PAD = N_CHUNKS * CH
PAD = E_PAD - E

NC, NS = 2, 16
NW = NC * NS

HALF = N_OUT // 2
ACC = 51200
DUMMY = N_OUT
ZB = 1600

_mesh = plsc.VectorSubcoreMesh(core_axis_name="c", subcore_axis_name="s")
_sc_params = pltpu.CompilerParams(use_tc_tiling_on_sc=False)


@functools.partial(
    pl.kernel,
    mesh=_mesh,
    compiler_params=_sc_params,
    out_type=jax.ShapeDtypeStruct((E_PAD, CIN), jnp.float32),
    scratch_types=[
        pltpu.VMEM((SUB, 128), jnp.int32),
        pltpu.VMEM((CH, CIN), jnp.float32),
        pltpu.SemaphoreType.DMA,
    ],
)
def _sc_gather(feats_hbm, idx_hbm, g_hbm, idx_v, rows_v, sem):
    wid = lax.axis_index("s") * NC + lax.axis_index("c")
    per = (N_CHUNKS + NW - 1) // NW

    @pl.loop(0, per)
    def _(j):
        c = wid * per + j

        @pl.when(c < N_CHUNKS)
        def _():
            pltpu.sync_copy(idx_hbm.at[pl.ds(c * SUB, SUB), :], idx_v)
            copies = []
            for u in range(SUB):
                copies.append(
                    pltpu.async_copy(
                        feats_hbm.at[idx_v.at[u]],
                        rows_v.at[pl.ds(u * 128, 128)],
                        sem,
                    )
                )
            for cp in copies:
                cp.wait()
            pltpu.sync_copy(rows_v, g_hbm.at[pl.ds(c * CH, CH), :])


def _mm_body(g_ref, w_ref, t_ref):
    t_ref[...] = lax.dot_general(
        g_ref[...],
        w_ref[0],
        (((1,), (0,)), ((), ())),
        preferred_element_type=jnp.float32,
        precision=lax.Precision.HIGHEST,
    )


_BS = 2000


def _tc_matmul(g, w):
    return pl.pallas_call(
        _mm_body,
        grid=(KV, S // _BS),
        in_specs=[
            pl.BlockSpec((_BS, CIN), lambda k, b: (k * (S // _BS) + b, 0)),
            pl.BlockSpec((1, CIN, COUT), lambda k, b: (k, 0, 0)),
        ],
        out_specs=pl.BlockSpec((_BS, COUT), lambda k, b: (k * (S // _BS) + b, 0)),
        out_shape=jax.ShapeDtypeStruct((E_PAD, COUT), jnp.float32),
    )(g, w)


@functools.partial(
    pl.kernel,
    mesh=_mesh,
    compiler_params=_sc_params,
    out_type=jax.ShapeDtypeStruct((N_OUT, 4, 16), jnp.float32),
    scratch_types=[
        pltpu.VMEM((SUB, 128), jnp.int32),
        pltpu.VMEM((CH, 1, 16), jnp.float32),
        pltpu.VMEM((ZB, 1, 16), jnp.float32),
        pltpu.VMEM_SHARED((ACC, 1, 16), jnp.float32),
        pltpu.SemaphoreType.DMA,
    ],
)
def _sc_scatter(t_hbm, oidx_hbm, out_hbm, oidx_v, tv_v, zb_v, acc_sh, sem):
    cid = lax.axis_index("c")
    sid = lax.axis_index("s")
    per = (N_CHUNKS + NS - 1) // NS

    @pl.loop(0, ZB)
    def _(i):
        zb_v[i, 0, :] = jnp.zeros((16,), jnp.float32)

    rows_per_sub = ACC // NS

    for cb_i in range(2):
        cb = cid * 2 + cb_i
        for h in range(2):
            lo = h * HALF

            @pl.loop(0, rows_per_sub // ZB)
            def _(z):
                pltpu.sync_copy(
                    zb_v, acc_sh.at[pl.ds(sid * rows_per_sub + z * ZB, ZB)]
                )

            plsc.subcore_barrier()

            @pl.loop(0, per)
            def _(j):
                c = sid * per + j

                @pl.when(c < N_CHUNKS)
                def _():
                    pltpu.sync_copy(oidx_hbm.at[pl.ds(c * SUB, SUB), :], oidx_v)
                    pltpu.sync_copy(
                        t_hbm.at[pl.ds(c * CH, CH), pl.ds(cb, 1), :], tv_v
                    )

                    @pl.loop(0, SUB)
                    def _(u):
                        @pl.loop(0, 8)
                        def _(l):
                            v = oidx_v[u, pl.ds(l * 16, 16)] - lo
                            ok = (v >= 0) & (v < HALF)
                            oidx_v[u, pl.ds(l * 16, 16)] = jnp.where(
                                ok, v, HALF
                            )

                    for u in range(SUB):
                        pltpu.sync_copy(
                            tv_v.at[pl.ds(u * 128, 128)],
                            acc_sh.at[oidx_v.at[u]],
                            add=True,
                        )

            plsc.subcore_barrier()

            pltpu.sync_copy(
                acc_sh.at[pl.ds(sid * 3125, 3125)],
                out_hbm.at[pl.ds(lo + sid * 3125, 3125), pl.ds(cb, 1), :],
            )

            plsc.subcore_barrier()


def kernel(feats, nbmaps, kernel):
    w = kernel
    in_idx = jnp.concatenate(
        [nbmaps[:, 0], jnp.zeros((PAD,), jnp.int32)]
    ).reshape(N_CHUNKS * SUB, 128)
    out_idx = jnp.concatenate(
        [nbmaps[:, 1], jnp.full((PAD,), DUMMY, jnp.int32)]
    ).reshape(N_CHUNKS * SUB, 128)

    g = _sc_gather(feats, in_idx)
    t = _tc_matmul(g, w)
    out3 = _sc_scatter(t.reshape(E_PAD, 4, 16), out_idx)
    return out3.reshape(N_OUT, COUT)


# speedup vs baseline: 1.0056x; 1.0056x over previous
import functools

import jax
import jax.numpy as jnp
from jax import lax
from jax.experimental import pallas as pl
from jax.experimental.pallas import tpu as pltpu
from jax.experimental.pallas import tpu_sc as plsc

N_IN = 100000
N_OUT = 100000
KV = 27
S = 60000
CIN = 32
COUT = 64

E = KV * S
CH = 1024
SUB = CH // 128
N_CHUNKS = (E + CH - 1) // CH
E_PAD = N_CHUNKS * CH
PAD = E_PAD - E

NC, NS = 2, 16
NW = NC * NS

HALF = N_OUT // 2
ACC = 51200
DUMMY = N_OUT
ZB = 1600

_mesh = plsc.VectorSubcoreMesh(core_axis_name="c", subcore_axis_name="s")
_sc_params = pltpu.CompilerParams(use_tc_tiling_on_sc=False)


@functools.partial(
    pl.kernel,
    mesh=_mesh,
    compiler_params=_sc_params,
    out_type=jax.ShapeDtypeStruct((E_PAD, CIN), jnp.float32),
    scratch_types=[
        pltpu.SemaphoreType.DMA,
    ],
)
def _sc_gather(feats_hbm, idx_hbm, g_hbm, sem):
    def body(idx_b, g_b):
        copies = []
        for u in range(SUB):
            copies.append(
                pltpu.async_copy(
                    feats_hbm.at[idx_b.at[u]],
                    g_b.at[pl.ds(u * 128, 128)],
                    sem,
                )
            )
        for cp in copies:
            cp.wait()

    pltpu.emit_pipeline(
        body,
        grid=(N_CHUNKS,),
        in_specs=[pl.BlockSpec((SUB, 128), lambda i: (i, 0))],
        out_specs=[pl.BlockSpec((CH, CIN), lambda i: (i, 0))],
        core_axis_name=("c", "s"),
        dimension_semantics=(pltpu.PARALLEL,),
    )(idx_hbm, g_hbm)


def _mm_body(g_ref, w_ref, t_ref):
    t_ref[...] = lax.dot_general(
        g_ref[...],
        w_ref[0],
        (((1,), (0,)), ((), ())),
        preferred_element_type=jnp.float32,
        precision=lax.Precision.HIGHEST,
    )


_BS = 2000


def _tc_matmul(g, w):
    return pl.pallas_call(
        _mm_body,
        grid=(KV, S // _BS),
        in_specs=[
            pl.BlockSpec((_BS, CIN), lambda k, b: (k * (S // _BS) + b, 0)),
            pl.BlockSpec((1, CIN, COUT), lambda k, b: (k, 0, 0)),
        ],
        out_specs=pl.BlockSpec((_BS, COUT), lambda k, b: (k * (S // _BS) + b, 0)),
        out_shape=jax.ShapeDtypeStruct((E_PAD, COUT), jnp.float32),
    )(g, w)


@functools.partial(
    pl.kernel,
    mesh=_mesh,
    compiler_params=_sc_params,
    out_type=jax.ShapeDtypeStruct((N_OUT, 4, 16), jnp.float32),
    scratch_types=[
        pltpu.VMEM((ZB, 1, 16), jnp.float32),
        pltpu.VMEM_SHARED((ACC, 1, 16), jnp.float32),
        pltpu.SemaphoreType.DMA,
        pltpu.SemaphoreType.DMA,
    ],
)
def _sc_scatter(t_hbm, oidx_hbm, out_hbm, zb_v, acc_sh, zsem, ssem):
    cid = lax.axis_index("c")
    sid = lax.axis_index("s")

    @pl.loop(0, ZB)
    def _(i):
        zb_v[i, 0, :] = jnp.zeros((16,), jnp.float32)

    rows_per_sub = ACC // NS

    for cb_i in range(2):
        cb = cid * 2 + cb_i
        for h in range(2):
            lo = h * HALF

            @pl.loop(0, rows_per_sub // ZB)
            def _(z):
                pltpu.sync_copy(
                    zb_v, acc_sh.at[pl.ds(sid * rows_per_sub + z * ZB, ZB)]
                )

            plsc.subcore_barrier()

            def body(oidx_b, tv_b):
                @pl.loop(0, SUB)
                def _(u):
                    @pl.loop(0, 8)
                    def _(l):
                        v = oidx_b[u, pl.ds(l * 16, 16)] - lo
                        ok = (v >= 0) & (v < HALF)
                        oidx_b[u, pl.ds(l * 16, 16)] = jnp.where(ok, v, HALF)

                copies = []
                for u in range(SUB):
                    copies.append(
                        pltpu.async_copy(
                            tv_b.at[pl.ds(u * 128, 128)],
                            acc_sh.at[oidx_b.at[u]],
                            ssem,
                            add=True,
                        )
                    )
                for cp in copies:
                    cp.wait()

            pltpu.emit_pipeline(
                body,
                grid=(N_CHUNKS,),
                in_specs=[
                    pl.BlockSpec((SUB, 128), lambda i: (i, 0)),
                    pl.BlockSpec((CH, 1, 16), lambda i: (i, 0, 0)),
                ],
                core_axis_name="s",
                dimension_semantics=(pltpu.PARALLEL,),
            )(oidx_hbm, t_hbm.at[:, pl.ds(cb, 1), :])

            plsc.subcore_barrier()

            pltpu.sync_copy(
                acc_sh.at[pl.ds(sid * 3125, 3125)],
                out_hbm.at[pl.ds(lo + sid * 3125, 3125), pl.ds(cb, 1), :],
            )

            plsc.subcore_barrier()


def kernel(feats, nbmaps, kernel):
    w = kernel
    in_idx = jnp.concatenate(
        [nbmaps[:, 0], jnp.zeros((PAD,), jnp.int32)]
    ).reshape(N_CHUNKS * SUB, 128)
    out_idx = jnp.concatenate(
        [nbmaps[:, 1], jnp.full((PAD,), DUMMY, jnp.int32)]
    ).reshape(N_CHUNKS * SUB, 128)

    g = _sc_gather(feats, in_idx)
    t = _tc_matmul(g, w)
    out3 = _sc_scatter(t.reshape(E_PAD, 4, 16), out_idx)
    return out3.reshape(N_OUT, COUT)


# speedup vs baseline: 1.5142x; 1.5058x over previous
import functools

import jax
import jax.numpy as jnp
from jax import lax
from jax.experimental import pallas as pl
from jax.experimental.pallas import tpu as pltpu
from jax.experimental.pallas import tpu_sc as plsc

N_IN = 100000
N_OUT = 100000
KV = 27
S = 60000
CIN = 32
COUT = 64

E = KV * S
CH = 1024
SUB = CH // 128
N_CHUNKS = (E + CH - 1) // CH
E_PAD = N_CHUNKS * CH
PAD = E_PAD - E

NC, NS = 2, 16
NW = NC * NS

HALF = N_OUT // 2
ACC = 51200
DUMMY = N_OUT
ZB = 1600

_mesh = plsc.VectorSubcoreMesh(core_axis_name="c", subcore_axis_name="s")
_sc_params = pltpu.CompilerParams(use_tc_tiling_on_sc=False)


@functools.partial(
    pl.kernel,
    mesh=_mesh,
    compiler_params=_sc_params,
    out_type=jax.ShapeDtypeStruct((E_PAD, CIN), jnp.float32),
    scratch_types=[
        pltpu.SemaphoreType.DMA,
    ],
)
def _sc_gather(feats_hbm, idx_hbm, g_hbm, sem):
    def body(idx_b, g_b):
        copies = []
        for u in range(SUB):
            copies.append(
                pltpu.async_copy(
                    feats_hbm.at[idx_b.at[u]],
                    g_b.at[pl.ds(u * 128, 128)],
                    sem,
                )
            )
        for cp in copies:
            cp.wait()

    pltpu.emit_pipeline(
        body,
        grid=(N_CHUNKS,),
        in_specs=[pl.BlockSpec((SUB, 128), lambda i: (i, 0))],
        out_specs=[pl.BlockSpec((CH, CIN), lambda i: (i, 0))],
        core_axis_name=("c", "s"),
        dimension_semantics=(pltpu.PARALLEL,),
    )(idx_hbm, g_hbm)


def _mm_body(g_ref, w_ref, t_ref):
    t_ref[...] = lax.dot_general(
        g_ref[...],
        w_ref[0],
        (((1,), (0,)), ((), ())),
        preferred_element_type=jnp.float32,
        precision=lax.Precision.HIGHEST,
    )


_BS = 2000


def _tc_matmul(g, w):
    return pl.pallas_call(
        _mm_body,
        grid=(KV, S // _BS),
        in_specs=[
            pl.BlockSpec((_BS, CIN), lambda k, b: (k * (S // _BS) + b, 0)),
            pl.BlockSpec((1, CIN, COUT), lambda k, b: (k, 0, 0)),
        ],
        out_specs=pl.BlockSpec((_BS, COUT), lambda k, b: (k * (S // _BS) + b, 0)),
        out_shape=jax.ShapeDtypeStruct((E_PAD, COUT), jnp.float32),
    )(g, w)


@functools.partial(
    pl.kernel,
    mesh=_mesh,
    compiler_params=_sc_params,
    out_type=jax.ShapeDtypeStruct((N_OUT, 4, 16), jnp.float32),
    scratch_types=[
        pltpu.VMEM((ZB, 1, 16), jnp.float32),
        pltpu.VMEM_SHARED((ACC, 1, 16), jnp.float32),
        pltpu.SemaphoreType.DMA,
        pltpu.SemaphoreType.DMA,
    ],
)
def _sc_scatter(t_hbm, oidx_hbm, out_hbm, zb_v, acc_sh, zsem, ssem):
    cid = lax.axis_index("c")
    sid = lax.axis_index("s")

    @pl.loop(0, ZB)
    def _(i):
        zb_v[i, 0, :] = jnp.zeros((16,), jnp.float32)

    rows_per_sub = ACC // NS

    for cb_i in range(2):
        cb = cid * 2 + cb_i
        for h in range(2):
            lo = h * HALF

            @pl.loop(0, rows_per_sub // ZB)
            def _(z):
                pltpu.sync_copy(
                    zb_v, acc_sh.at[pl.ds(sid * rows_per_sub + z * ZB, ZB)]
                )

            plsc.subcore_barrier()

            def body(oidx_b, tv_b):
                @pl.loop(0, SUB)
                def _(u):
                    @pl.loop(0, 8)
                    def _(l):
                        v = oidx_b[u, pl.ds(l * 16, 16)] - lo
                        ok = (v >= 0) & (v < HALF)
                        dummy = HALF + (v & 1023)
                        oidx_b[u, pl.ds(l * 16, 16)] = jnp.where(ok, v, dummy)

                copies = []
                for u in range(SUB):
                    copies.append(
                        pltpu.async_copy(
                            tv_b.at[pl.ds(u * 128, 128)],
                            acc_sh.at[oidx_b.at[u]],
                            ssem,
                            add=True,
                        )
                    )
                for cp in copies:
                    cp.wait()

            pltpu.emit_pipeline(
                body,
                grid=(N_CHUNKS,),
                in_specs=[
                    pl.BlockSpec((SUB, 128), lambda i: (i, 0)),
                    pl.BlockSpec((CH, 1, 16), lambda i: (i, 0, 0)),
                ],
                core_axis_name="s",
                dimension_semantics=(pltpu.PARALLEL,),
            )(oidx_hbm, t_hbm.at[:, pl.ds(cb, 1), :])

            plsc.subcore_barrier()

            pltpu.sync_copy(
                acc_sh.at[pl.ds(sid * 3125, 3125)],
                out_hbm.at[pl.ds(lo + sid * 3125, 3125), pl.ds(cb, 1), :],
            )

            plsc.subcore_barrier()


def kernel(feats, nbmaps, kernel):
    w = kernel
    in_idx = jnp.concatenate(
        [nbmaps[:, 0], jnp.zeros((PAD,), jnp.int32)]
    ).reshape(N_CHUNKS * SUB, 128)
    out_idx = jnp.concatenate(
        [nbmaps[:, 1], jnp.full((PAD,), DUMMY, jnp.int32)]
    ).reshape(N_CHUNKS * SUB, 128)

    g = _sc_gather(feats, in_idx)
    t = _tc_matmul(g, w)
    out3 = _sc_scatter(t.reshape(E_PAD, 4, 16), out_idx)
    return out3.reshape(N_OUT, COUT)


# speedup vs baseline: 5.1925x; 3.4291x over previous
import functools

import jax
import jax.numpy as jnp
from jax import lax
from jax.experimental import pallas as pl
from jax.experimental.pallas import tpu as pltpu
from jax.experimental.pallas import tpu_sc as plsc

N_IN = 100000
N_OUT = 100000
KV = 27
S = 60000
CIN = 32
COUT = 64

E = KV * S
CH = 1024
SUB = CH // 128
N_CHUNKS = (E + CH - 1) // CH
E_PAD = N_CHUNKS * CH
PAD = E_PAD - E

NC, NS = 2, 16
NW = NC * NS

HALF = N_OUT // 2
ACC = 51200
DUMMY = N_OUT
ZB = 1600

_mesh = plsc.VectorSubcoreMesh(core_axis_name="c", subcore_axis_name="s")
_sc_params = pltpu.CompilerParams(use_tc_tiling_on_sc=False)


@functools.partial(
    pl.kernel,
    mesh=_mesh,
    compiler_params=_sc_params,
    out_type=jax.ShapeDtypeStruct((E_PAD, CIN), jnp.float32),
    scratch_types=[
        pltpu.SemaphoreType.DMA,
    ],
)
def _sc_gather(feats_hbm, idx_hbm, g_hbm, sem):
    def body(idx_b, g_b):
        copies = []
        for u in range(SUB):
            copies.append(
                pltpu.async_copy(
                    feats_hbm.at[idx_b.at[u]],
                    g_b.at[pl.ds(u * 128, 128)],
                    sem,
                )
            )
        for cp in copies:
            cp.wait()

    pltpu.emit_pipeline(
        body,
        grid=(N_CHUNKS,),
        in_specs=[pl.BlockSpec((SUB, 128), lambda i: (i, 0))],
        out_specs=[pl.BlockSpec((CH, CIN), lambda i: (i, 0))],
        core_axis_name=("c", "s"),
        dimension_semantics=(pltpu.PARALLEL,),
    )(idx_hbm, g_hbm)


def _mm_body(g_ref, w_ref, t_ref):
    t_ref[...] = lax.dot_general(
        g_ref[...],
        w_ref[0],
        (((1,), (0,)), ((), ())),
        preferred_element_type=jnp.float32,
    )


_BS4 = 1000


def _tc_matmul(g4, w4):
    nb = S // (4 * _BS4)
    return pl.pallas_call(
        _mm_body,
        grid=(KV, nb),
        in_specs=[
            pl.BlockSpec((_BS4, 4 * CIN), lambda k, b: (k * nb + b, 0)),
            pl.BlockSpec((1, 4 * CIN, 4 * COUT), lambda k, b: (k, 0, 0)),
        ],
        out_specs=pl.BlockSpec((_BS4, 4 * COUT), lambda k, b: (k * nb + b, 0)),
        out_shape=jax.ShapeDtypeStruct((E_PAD // 4, 4 * COUT), jnp.float32),
    )(g4, w4)


@functools.partial(
    pl.kernel,
    mesh=_mesh,
    compiler_params=_sc_params,
    out_type=jax.ShapeDtypeStruct((N_OUT, 4, 16), jnp.float32),
    scratch_types=[
        pltpu.VMEM((ZB, 1, 16), jnp.float32),
        pltpu.VMEM_SHARED((ACC, 1, 16), jnp.float32),
        pltpu.SemaphoreType.DMA,
        pltpu.SemaphoreType.DMA,
    ],
)
def _sc_scatter(t_hbm, oidx_hbm, out_hbm, zb_v, acc_sh, zsem, ssem):
    cid = lax.axis_index("c")
    sid = lax.axis_index("s")

    @pl.loop(0, ZB)
    def _(i):
        zb_v[i, 0, :] = jnp.zeros((16,), jnp.float32)

    rows_per_sub = ACC // NS

    for cb_i in range(2):
        cb = cid * 2 + cb_i
        for h in range(2):
            lo = h * HALF

            @pl.loop(0, rows_per_sub // ZB)
            def _(z):
                pltpu.sync_copy(
                    zb_v, acc_sh.at[pl.ds(sid * rows_per_sub + z * ZB, ZB)]
                )

            plsc.subcore_barrier()

            def body(oidx_b, tv_b):
                @pl.loop(0, SUB)
                def _(u):
                    @pl.loop(0, 8)
                    def _(l):
                        v = oidx_b[u, pl.ds(l * 16, 16)] - lo
                        ok = (v >= 0) & (v < HALF)
                        dummy = HALF + (v & 1023)
                        oidx_b[u, pl.ds(l * 16, 16)] = jnp.where(ok, v, dummy)

                copies = []
                for u in range(SUB):
                    copies.append(
                        pltpu.async_copy(
                            tv_b.at[pl.ds(u * 128, 128)],
                            acc_sh.at[oidx_b.at[u]],
                            ssem,
                            add=True,
                        )
                    )
                for cp in copies:
                    cp.wait()

            pltpu.emit_pipeline(
                body,
                grid=(N_CHUNKS,),
                in_specs=[
                    pl.BlockSpec((SUB, 128), lambda i: (i, 0)),
                    pl.BlockSpec((CH, 1, 16), lambda i: (i, 0, 0)),
                ],
                core_axis_name="s",
                dimension_semantics=(pltpu.PARALLEL,),
            )(oidx_hbm, t_hbm.at[:, pl.ds(cb, 1), :])

            plsc.subcore_barrier()

            pltpu.sync_copy(
                acc_sh.at[pl.ds(sid * 3125, 3125)],
                out_hbm.at[pl.ds(lo + sid * 3125, 3125), pl.ds(cb, 1), :],
            )

            plsc.subcore_barrier()


def kernel(feats, nbmaps, kernel):
    w = kernel
    in_idx = jnp.concatenate(
        [nbmaps[:, 0], jnp.zeros((PAD,), jnp.int32)]
    ).reshape(N_CHUNKS * SUB, 128)
    out_idx = jnp.concatenate(
        [nbmaps[:, 1], jnp.full((PAD,), DUMMY, jnp.int32)]
    ).reshape(N_CHUNKS * SUB, 128)

    w4 = jax.vmap(lambda wk: jnp.kron(jnp.eye(4, dtype=wk.dtype), wk))(w)
    g = _sc_gather(feats, in_idx)
    t = _tc_matmul(g.reshape(E_PAD // 4, 4 * CIN), w4)
    out3 = _sc_scatter(t.reshape(E_PAD, 4, 16), out_idx)
    return out3.reshape(N_OUT, COUT)


# speedup vs baseline: 5.2908x; 1.0189x over previous
import functools

import jax
import jax.numpy as jnp
from jax import lax
from jax.experimental import pallas as pl
from jax.experimental.pallas import tpu as pltpu
from jax.experimental.pallas import tpu_sc as plsc

N_IN = 100000
N_OUT = 100000
KV = 27
S = 60000
CIN = 32
COUT = 64

E = KV * S
CH = 1024
SUB = CH // 128
CH3 = 2048
SUB3 = CH3 // 128
E_PAD = 1622016
N_CHUNKS = E_PAD // CH
N_CHUNKS3 = E_PAD // CH3
PAD = E_PAD - E

NC, NS = 2, 16
NW = NC * NS

HALF = N_OUT // 2
ACC = 51200
DUMMY = N_OUT
ZB = 400

_mesh = plsc.VectorSubcoreMesh(core_axis_name="c", subcore_axis_name="s")
_sc_params = pltpu.CompilerParams(use_tc_tiling_on_sc=False)


@functools.partial(
    pl.kernel,
    mesh=_mesh,
    compiler_params=_sc_params,
    out_type=jax.ShapeDtypeStruct((E_PAD, CIN), jnp.float32),
    scratch_types=[
        pltpu.SemaphoreType.DMA,
    ],
)
def _sc_gather(feats_hbm, idx_hbm, g_hbm, sem):
    def body(idx_b, g_b):
        copies = []
        for u in range(SUB):
            copies.append(
                pltpu.async_copy(
                    feats_hbm.at[idx_b.at[u]],
                    g_b.at[pl.ds(u * 128, 128)],
                    sem,
                )
            )
        for cp in copies:
            cp.wait()

    pltpu.emit_pipeline(
        body,
        grid=(N_CHUNKS,),
        in_specs=[pl.BlockSpec((SUB, 128), lambda i: (i, 0))],
        out_specs=[pl.BlockSpec((CH, CIN), lambda i: (i, 0))],
        core_axis_name=("c", "s"),
        dimension_semantics=(pltpu.PARALLEL,),
    )(idx_hbm, g_hbm)


def _mm_body(g_ref, w_ref, t_ref):
    t_ref[...] = lax.dot_general(
        g_ref[...],
        w_ref[0],
        (((1,), (0,)), ((), ())),
        preferred_element_type=jnp.float32,
    )


_BS4 = 1000


def _tc_matmul(g4, w4):
    nb = S // (4 * _BS4)
    return pl.pallas_call(
        _mm_body,
        grid=(KV, nb),
        in_specs=[
            pl.BlockSpec((_BS4, 4 * CIN), lambda k, b: (k * nb + b, 0)),
            pl.BlockSpec((1, 4 * CIN, 4 * COUT), lambda k, b: (k, 0, 0)),
        ],
        out_specs=pl.BlockSpec((_BS4, 4 * COUT), lambda k, b: (k * nb + b, 0)),
        out_shape=jax.ShapeDtypeStruct((E_PAD // 4, 4 * COUT), jnp.float32),
    )(g4, w4)


@functools.partial(
    pl.kernel,
    mesh=_mesh,
    compiler_params=_sc_params,
    out_type=jax.ShapeDtypeStruct((N_OUT, 4, 16), jnp.float32),
    scratch_types=[
        pltpu.VMEM((ZB, 1, 16), jnp.float32),
        pltpu.VMEM_SHARED((ACC, 1, 16), jnp.float32),
        pltpu.SemaphoreType.DMA,
        pltpu.SemaphoreType.DMA,
    ],
)
def _sc_scatter(t_hbm, oidx_hbm, out_hbm, zb_v, acc_sh, zsem, ssem):
    cid = lax.axis_index("c")
    sid = lax.axis_index("s")

    @pl.loop(0, ZB)
    def _(i):
        zb_v[i, 0, :] = jnp.zeros((16,), jnp.float32)

    rows_per_sub = ACC // NS

    for cb_i in range(2):
        cb = cid * 2 + cb_i
        for h in range(2):
            lo = h * HALF

            @pl.loop(0, rows_per_sub // ZB)
            def _(z):
                pltpu.sync_copy(
                    zb_v, acc_sh.at[pl.ds(sid * rows_per_sub + z * ZB, ZB)]
                )

            plsc.subcore_barrier()

            def body(oidx_b, tv_b):
                @pl.loop(0, SUB3)
                def _(u):
                    @pl.loop(0, 8)
                    def _(l):
                        v = oidx_b[u, pl.ds(l * 16, 16)] - lo
                        ok = (v >= 0) & (v < HALF)
                        dummy = HALF + (v & 1023)
                        oidx_b[u, pl.ds(l * 16, 16)] = jnp.where(ok, v, dummy)

                copies = []
                for u in range(SUB3):
                    copies.append(
                        pltpu.async_copy(
                            tv_b.at[pl.ds(u * 128, 128)],
                            acc_sh.at[oidx_b.at[u]],
                            ssem,
                            add=True,
                        )
                    )
                for cp in copies:
                    cp.wait()

            pltpu.emit_pipeline(
                body,
                grid=(N_CHUNKS3,),
                in_specs=[
                    pl.BlockSpec((SUB3, 128), lambda i: (i, 0)),
                    pl.BlockSpec((CH3, 1, 16), lambda i: (i, 0, 0)),
                ],
                core_axis_name="s",
                dimension_semantics=(pltpu.PARALLEL,),
            )(oidx_hbm, t_hbm.at[:, pl.ds(cb, 1), :])

            plsc.subcore_barrier()

            pltpu.sync_copy(
                acc_sh.at[pl.ds(sid * 3125, 3125)],
                out_hbm.at[pl.ds(lo + sid * 3125, 3125), pl.ds(cb, 1), :],
            )

            plsc.subcore_barrier()


def kernel(feats, nbmaps, kernel):
    w = kernel
    in_idx = jnp.concatenate(
        [nbmaps[:, 0], jnp.zeros((PAD,), jnp.int32)]
    ).reshape(N_CHUNKS * SUB, 128)
    out_idx = jnp.concatenate(
        [nbmaps[:, 1], jnp.full((PAD,), DUMMY, jnp.int32)]
    ).reshape(N_CHUNKS3 * SUB3, 128)

    w4 = jax.vmap(lambda wk: jnp.kron(jnp.eye(4, dtype=wk.dtype), wk))(w)
    g = _sc_gather(feats, in_idx)
    t = _tc_matmul(g.reshape(E_PAD // 4, 4 * CIN), w4)
    out3 = _sc_scatter(t.reshape(E_PAD, 4, 16), out_idx)
    return out3.reshape(N_OUT, COUT)
